# baseline (speedup 1.0000x reference)
import functools

import jax
import jax.numpy as jnp
from jax.experimental import pallas as pl
from jax.experimental.pallas import tpu as pltpu

LAGS = (1, 2, 4, 8, 12)
MAXLAG = 12
CLAMP_MIN, CLAMP_MAX = -2.5, 2.5


def _body(params_ref, v_ref, e_ref, w1_ref, w2_ref, o_ref, carry_ref, *, tb, d_f, d_g):
    t = pl.program_id(1)
    v = v_ref[0]
    a = jnp.tanh(
        jnp.dot(v, w1_ref[:], preferred_element_type=jnp.float32,
                precision=jax.lax.Precision.HIGHEST)
    )

    s = jnp.mean(v, axis=1, keepdims=True)

    @pl.when(t == 0)
    def _():
        carry_ref[:] = jnp.broadcast_to(s[0:1], (MAXLAG, 1))

    hist = jnp.concatenate([carry_ref[:], s], axis=0)
    mu = jnp.full_like(s, params_ref[len(LAGS)])
    ls = jnp.full_like(s, params_ref[2 * len(LAGS) + 1])
    for i, lag in enumerate(LAGS):
        sl = hist[MAXLAG - lag:MAXLAG - lag + tb]
        mu = mu + params_ref[i] * sl
        ls = ls + params_ref[len(LAGS) + 1 + i] * sl
    h = mu + jnp.exp(ls) * e_ref[0]
    carry_ref[:] = s[tb - MAXLAG:]

    col = jax.lax.broadcasted_iota(jnp.int32, (1, d_f + d_g), 1)
    m = a * jnp.where(col >= d_f, h, 1.0)
    o_ref[0, 0] = jnp.clip(
        jnp.dot(m, w2_ref[:], preferred_element_type=jnp.float32,
                precision=jax.lax.Precision.HIGHEST),
        CLAMP_MIN, CLAMP_MAX,
    )


@jax.jit
def kernel(visible, W1f, W2f, W1g, W2g, w_mu, b_mu, w_ls, b_ls):
    B, T, N = visible.shape
    d_f = W1f.shape[1]
    d_g = W1g.shape[1]
    tb = 512

    eps = jax.random.normal(jax.random.key(42), (1, B, T), jnp.float32)
    eps = eps[0].reshape(B, T, 1)
    w1 = jnp.concatenate([W1f, W1g], axis=1)
    w2 = jnp.concatenate([W2f, W2g], axis=0)
    params = jnp.concatenate([
        w_mu, b_mu[None], w_ls, b_ls[None]
    ]).astype(jnp.float32)

    out = pl.pallas_call(
        functools.partial(_body, tb=tb, d_f=d_f, d_g=d_g),
        grid=(B, T // tb),
        in_specs=[
            pl.BlockSpec(memory_space=pltpu.SMEM),
            pl.BlockSpec((1, tb, N), lambda b, t: (b, t, 0)),
            pl.BlockSpec((1, tb, 1), lambda b, t: (b, t, 0)),
            pl.BlockSpec((N, d_f + d_g), lambda b, t: (0, 0)),
            pl.BlockSpec((d_f + d_g, N), lambda b, t: (0, 0)),
        ],
        out_specs=pl.BlockSpec((1, 1, tb, N), lambda b, t: (0, b, t, 0)),
        out_shape=jax.ShapeDtypeStruct((1, B, T, N), jnp.float32),
        scratch_shapes=[pltpu.VMEM((MAXLAG, 1), jnp.float32)],
        compiler_params=pltpu.CompilerParams(
            dimension_semantics=("parallel", "arbitrary"),
        ),
    )(params, visible, eps, w1, w2)
    return out


# speedup vs baseline: 1.9913x; 1.9913x over previous
import functools

import jax
import jax.numpy as jnp
from jax.experimental import pallas as pl
from jax.experimental.pallas import tpu as pltpu

LAGS = (1, 2, 4, 8, 12)
MAXLAG = 12
CLAMP_MIN, CLAMP_MAX = -2.5, 2.5


def _body(params_ref, v_ref, e_ref, w1_ref, w2_ref, o_ref,
          a_ref, sprev_ref, carry_ref, *, tb, nt, d_f, d_g):
    t = pl.program_id(1)

    @pl.when(t > 0)
    def _():
        hist = jnp.concatenate([carry_ref[:], sprev_ref[:]], axis=1)
        mu = jnp.full((1, tb), params_ref[len(LAGS)], jnp.float32)
        ls = jnp.full((1, tb), params_ref[2 * len(LAGS) + 1], jnp.float32)
        for i, lag in enumerate(LAGS):
            sl = hist[:, MAXLAG - lag:MAXLAG - lag + tb]
            mu = mu + params_ref[i] * sl
            ls = ls + params_ref[len(LAGS) + 1 + i] * sl
        h = jnp.transpose(mu + jnp.exp(ls) * e_ref[0])
        carry_ref[:] = hist[:, tb:]
        col = jax.lax.broadcasted_iota(jnp.int32, (1, d_f + d_g), 1)
        m = a_ref[:] * jnp.where(col >= d_f, h, 1.0)
        o_ref[0, 0] = jnp.clip(
            jnp.dot(m, w2_ref[:], preferred_element_type=jnp.float32,
                    precision=jax.lax.Precision.HIGHEST),
            CLAMP_MIN, CLAMP_MAX,
        )

    @pl.when(t < nt)
    def _():
        v = v_ref[0]
        a_ref[:] = jnp.tanh(
            jnp.dot(v, w1_ref[:], preferred_element_type=jnp.float32,
                    precision=jax.lax.Precision.HIGHEST)
        )
        s = jnp.transpose(jnp.mean(v, axis=1, keepdims=True))

        @pl.when(t == 0)
        def _():
            carry_ref[:] = jnp.broadcast_to(s[:, 0:1], (1, MAXLAG))

        sprev_ref[:] = s


@jax.jit
def kernel(visible, W1f, W2f, W1g, W2g, w_mu, b_mu, w_ls, b_ls):
    B, T, N = visible.shape
    d_f = W1f.shape[1]
    d_g = W1g.shape[1]
    tb = 512
    nt = T // tb

    eps = jax.random.normal(jax.random.key(42), (1, B, T), jnp.float32)
    eps = eps.reshape(B, 1, T)
    w1 = jnp.concatenate([W1f, W1g], axis=1)
    w2 = jnp.concatenate([W2f, W2g], axis=0)
    params = jnp.concatenate([
        w_mu, b_mu[None], w_ls, b_ls[None]
    ]).astype(jnp.float32)

    out = pl.pallas_call(
        functools.partial(_body, tb=tb, nt=nt, d_f=d_f, d_g=d_g),
        grid=(B, nt + 1),
        in_specs=[
            pl.BlockSpec(memory_space=pltpu.SMEM),
            pl.BlockSpec((1, tb, N), lambda b, t: (b, jnp.minimum(t, nt - 1), 0)),
            pl.BlockSpec((1, 1, tb), lambda b, t: (b, 0, jnp.maximum(t - 1, 0))),
            pl.BlockSpec((N, d_f + d_g), lambda b, t: (0, 0)),
            pl.BlockSpec((d_f + d_g, N), lambda b, t: (0, 0)),
        ],
        out_specs=pl.BlockSpec((1, 1, tb, N), lambda b, t: (0, b, jnp.maximum(t - 1, 0), 0)),
        out_shape=jax.ShapeDtypeStruct((1, B, T, N), jnp.float32),
        scratch_shapes=[
            pltpu.VMEM((tb, d_f + d_g), jnp.float32),
            pltpu.VMEM((1, tb), jnp.float32),
            pltpu.VMEM((1, MAXLAG), jnp.float32),
        ],
        compiler_params=pltpu.CompilerParams(
            dimension_semantics=("parallel", "arbitrary"),
        ),
    )(params, visible, eps, w1, w2)
    return out
